# baseline (speedup 1.0000x reference)
import functools

import jax
import jax.numpy as jnp
from jax import lax
from jax.experimental import pallas as pl
from jax.experimental.pallas import tpu as pltpu
from jax.experimental.pallas import tpu_sc as plsc

_N = 10000
_E = 320000
_C = 128
_R = 10

_NC = 2
_NS = 16
_NW = _NC * _NS
_EPW = _E // _NW
_B = 128
_NCHUNK = -(-_EPW // _B)
_EPAD = _NCHUNK * _B - _EPW
_RPS = 624
_RTAIL = _N - _RPS * _NS

_mesh = plsc.VectorSubcoreMesh(core_axis_name="c", subcore_axis_name="s")


@functools.partial(
    pl.kernel,
    out_type=jax.ShapeDtypeStruct((_NC * _N, _C), jnp.float32),
    mesh=_mesh,
    scratch_types=[
        pltpu.VMEM((_NCHUNK, _B), jnp.int32),
        pltpu.VMEM((2, _B), jnp.int32),
        pltpu.VMEM((2, _B), jnp.int32),
        pltpu.VMEM((_B,), jnp.int32),
        pltpu.VMEM((_B,), jnp.int32),
        pltpu.VMEM((_B, _C), jnp.float32),
        pltpu.VMEM((_B, _C), jnp.float32),
        pltpu.VMEM_SHARED((_N + 8, _C), jnp.float32),
        pltpu.SemaphoreType.DMA,
        pltpu.SemaphoreType.DMA,
        pltpu.SemaphoreType.DMA,
        pltpu.SemaphoreType.DMA,
        pltpu.SemaphoreType.DMA,
        pltpu.SemaphoreType.DMA,
    ],
)
def _sc_aggregate(scaled_hbm, te_hbm, head_hbm, out_hbm,
                  head_v, te0, te1, cidx0, cidx1, rows0, rows1, acc_sh,
                  tsem0, tsem1, gsem0, gsem1, ssem0, ssem1):
    core = lax.axis_index("c")
    sid = lax.axis_index("s")
    wid = sid * _NC + core

    def issue_te(jx, te, tsem):
        pltpu.async_copy(te_hbm.at[wid, jx], te, tsem)

    def wait_te(jx, te, tsem):
        pltpu.make_async_copy(te_hbm.at[wid, jx], te, tsem).wait()

    def compute_cidx(te, cidx):
        for g in range(_B // 16):
            sl = pl.ds(g * 16, 16)
            cidx[sl] = te[1, sl] * _N + te[0, sl]

    _H = _B // 2

    def issue_gather(cidx, rows, gsem):
        pltpu.async_copy(scaled_hbm.at[cidx.at[pl.ds(0, _H)]],
                         rows.at[pl.ds(0, _H)], gsem)
        pltpu.async_copy(scaled_hbm.at[cidx.at[pl.ds(_H, _H)]],
                         rows.at[pl.ds(_H, _H)], gsem)

    def wait_gather(cidx, rows, gsem):
        pltpu.make_async_copy(scaled_hbm.at[cidx.at[pl.ds(0, _H)]],
                              rows.at[pl.ds(0, _H)], gsem).wait()
        pltpu.make_async_copy(scaled_hbm.at[cidx.at[pl.ds(_H, _H)]],
                              rows.at[pl.ds(_H, _H)], gsem).wait()

    def issue_scatter(jx, rows, ssem):
        pltpu.async_copy(rows, acc_sh.at[head_v.at[jx]], ssem, add=True)

    def wait_scatter(jx, rows, ssem):
        pltpu.make_async_copy(rows, acc_sh.at[head_v.at[jx]], ssem).wait()

    pltpu.sync_copy(head_hbm.at[wid], head_v)
    pltpu.sync_copy(te_hbm.at[wid, 0], te0)
    compute_cidx(te0, cidx0)
    issue_gather(cidx0, rows0, gsem0)
    issue_te(1, te1, tsem1)

    def zero_body(i, carry):
        for c in range(_C // 16):
            rows1[i, pl.ds(c * 16, 16)] = jnp.zeros((16,), jnp.float32)
        return carry

    lax.fori_loop(0, _B, zero_body, 0)
    full, rem = _RPS // _B, _RPS % _B
    for k in range(full):
        pltpu.sync_copy(rows1, acc_sh.at[pl.ds(sid * _RPS + k * _B, _B)])
    if rem:
        pltpu.sync_copy(rows1.at[pl.ds(0, rem)],
                        acc_sh.at[pl.ds(sid * _RPS + full * _B, rem)])

    @pl.when(sid == _NS - 1)
    def _zero_tail():
        pltpu.sync_copy(rows1.at[pl.ds(0, _RTAIL)],
                        acc_sh.at[pl.ds(_RPS * _NS, _RTAIL)])

    plsc.subcore_barrier()

    _J2 = (_NCHUNK - 1) // 2

    def body(j2, carry):
        a = 2 * j2
        b = a + 1
        wait_te(b, te1, tsem1)

        @pl.when(j2 > 0)
        def _drain_prev_odd():
            wait_scatter(b - 2, rows1, ssem1)

        compute_cidx(te1, cidx1)
        issue_gather(cidx1, rows1, gsem1)
        wait_gather(cidx0, rows0, gsem0)
        issue_te(a + 2, te0, tsem0)
        issue_scatter(a, rows0, ssem0)

        wait_gather(cidx1, rows1, gsem1)
        wait_te(a + 2, te0, tsem0)
        wait_scatter(a, rows0, ssem0)
        compute_cidx(te0, cidx0)
        issue_gather(cidx0, rows0, gsem0)

        @pl.when(j2 < _J2 - 1)
        def _prefetch_next_odd():
            issue_te(b + 2, te1, tsem1)

        issue_scatter(b, rows1, ssem1)
        return carry

    lax.fori_loop(0, _J2, body, 0)

    last = _NCHUNK - 1
    wait_scatter(last - 1, rows1, ssem1)
    wait_gather(cidx0, rows0, gsem0)
    issue_scatter(last, rows0, ssem0)
    wait_scatter(last, rows0, ssem0)

    plsc.subcore_barrier()

    start = sid * _RPS
    pltpu.sync_copy(acc_sh.at[pl.ds(start, _RPS)],
                    out_hbm.at[pl.ds(core * _N + start, _RPS)])

    @pl.when(sid == _NS - 1)
    def _write_tail():
        pltpu.sync_copy(acc_sh.at[pl.ds(_RPS * _NS, _RTAIL)],
                        out_hbm.at[pl.ds(core * _N + _RPS * _NS, _RTAIL)])


_BLK = 2000


def _scale_body(emb_ref, w_ref, o_ref):
    o_ref[...] = emb_ref[...][None, :, :] * w_ref[...][:, None, :]


def _scale(emb, w):
    return pl.pallas_call(
        _scale_body,
        grid=(_N // _BLK,),
        in_specs=[
            pl.BlockSpec((_BLK, _C), lambda i: (i, 0)),
            pl.BlockSpec((_R, _C), lambda i: (0, 0)),
        ],
        out_specs=pl.BlockSpec((_R, _BLK, _C), lambda i: (0, i, 0)),
        out_shape=jax.ShapeDtypeStruct((_R, _N, _C), jnp.float32),
    )(emb, w)


def _combine_body(a_ref, b_ref, o_ref):
    o_ref[...] = a_ref[...] + b_ref[...]


_CBLK = 2000


def _combine(a, b):
    return pl.pallas_call(
        _combine_body,
        grid=(_N // _CBLK,),
        in_specs=[pl.BlockSpec((_CBLK, _C), lambda i: (i, 0))] * 2,
        out_specs=pl.BlockSpec((_CBLK, _C), lambda i: (i, 0)),
        out_shape=jax.ShapeDtypeStruct((_N, _C), jnp.float32),
    )(a, b)


def _pad_per_worker(x, fill):
    return jnp.pad(x.reshape(_NW, _EPW), ((0, 0), (0, _EPAD)),
                   constant_values=fill)


def kernel(all_emb, edge_index, edge_type, weight):
    scaled = _scale(all_emb, weight).reshape(_R * _N, _C)
    tail2 = _pad_per_worker(edge_index[1], 0).reshape(_NW, _NCHUNK, 1, _B)
    et2 = _pad_per_worker(edge_type, 0).reshape(_NW, _NCHUNK, 1, _B)
    te = jnp.concatenate([tail2, et2], axis=2)
    head3 = _pad_per_worker(edge_index[0], _N).reshape(_NW, _NCHUNK, _B)
    partial = _sc_aggregate(scaled, te, head3)
    return _combine(partial[:_N], partial[_N:])


# speedup vs baseline: 1.1580x; 1.1580x over previous
import functools

import jax
import jax.numpy as jnp
from jax import lax
from jax.experimental import pallas as pl
from jax.experimental.pallas import tpu as pltpu
from jax.experimental.pallas import tpu_sc as plsc

_N = 10000
_E = 320000
_C = 128
_R = 10

_NC = 2
_NS = 16
_NW = _NC * _NS
_EPW = _E // _NW
_B = 112
_NCHUNK = -(-_EPW // _B)
_EPAD = _NCHUNK * _B - _EPW
_RPS = 624
_RTAIL = _N - _RPS * _NS

_mesh = plsc.VectorSubcoreMesh(core_axis_name="c", subcore_axis_name="s")


@functools.partial(
    pl.kernel,
    out_type=jax.ShapeDtypeStruct((_NC * _N, _C), jnp.float32),
    mesh=_mesh,
    scratch_types=[
        pltpu.VMEM((3, _B), jnp.int32),
        pltpu.VMEM((3, _B), jnp.int32),
        pltpu.VMEM((3, _B), jnp.int32),
        pltpu.VMEM((_B,), jnp.int32),
        pltpu.VMEM((_B,), jnp.int32),
        pltpu.VMEM((_B,), jnp.int32),
        pltpu.VMEM((_B,), jnp.int32),
        pltpu.VMEM((_B,), jnp.int32),
        pltpu.VMEM((_B,), jnp.int32),
        pltpu.VMEM((_B, _C), jnp.float32),
        pltpu.VMEM((_B, _C), jnp.float32),
        pltpu.VMEM((_B, _C), jnp.float32),
        pltpu.VMEM_SHARED((_N + 8, _C), jnp.float32),
        pltpu.SemaphoreType.DMA,
        pltpu.SemaphoreType.DMA,
        pltpu.SemaphoreType.DMA,
        pltpu.SemaphoreType.DMA,
        pltpu.SemaphoreType.DMA,
        pltpu.SemaphoreType.DMA,
        pltpu.SemaphoreType.DMA,
        pltpu.SemaphoreType.DMA,
        pltpu.SemaphoreType.DMA,
    ],
)
def _sc_aggregate(scaled_hbm, te_hbm, out_hbm,
                  te0, te1, te2, ci0, ci1, ci2, hb0, hb1, hb2,
                  rw0, rw1, rw2, acc_sh,
                  ts0, ts1, ts2, gs0, gs1, gs2, ss0, ss1, ss2):
    core = lax.axis_index("c")
    sid = lax.axis_index("s")
    wid = sid * _NC + core

    te = (te0, te1, te2)
    ci = (ci0, ci1, ci2)
    hb = (hb0, hb1, hb2)
    rw = (rw0, rw1, rw2)
    ts = (ts0, ts1, ts2)
    gs = (gs0, gs1, gs2)
    ss = (ss0, ss1, ss2)

    def issue_te(jx, q):
        pltpu.async_copy(te_hbm.at[wid, jx], te[q], ts[q])

    def wait_te(jx, q):
        pltpu.make_async_copy(te_hbm.at[wid, jx], te[q], ts[q]).wait()

    def prep_idx(q):
        for g in range(_B // 16):
            sl = pl.ds(g * 16, 16)
            ci[q][sl] = te[q][1, sl] * _N + te[q][0, sl]
            hb[q][sl] = te[q][2, sl]

    def issue_gather(q):
        pltpu.async_copy(scaled_hbm.at[ci[q]], rw[q], gs[q])

    def wait_gather(q):
        pltpu.make_async_copy(scaled_hbm.at[ci[q]], rw[q], gs[q]).wait()

    def issue_scatter(q):
        pltpu.async_copy(rw[q], acc_sh.at[hb[q]], ss[q], add=True)

    def wait_scatter(q):
        pltpu.make_async_copy(rw[q], acc_sh.at[hb[q]], ss[q]).wait()

    def slot(j, p, q, scat_wait=True, do_gather=True, do_te=True):
        wait_gather(p)
        issue_scatter(p)
        if do_gather:
            wait_te(j + 2, q)
        if scat_wait:
            wait_scatter(q)
        if do_gather:
            prep_idx(q)
            issue_gather(q)
        if do_te:
            issue_te(j + 4, (p + 1) % 3)

    pltpu.sync_copy(te_hbm.at[wid, 0], te0)
    pltpu.sync_copy(te_hbm.at[wid, 1], te1)
    prep_idx(0)
    prep_idx(1)
    issue_gather(0)
    issue_gather(1)
    issue_te(2, 2)
    issue_te(3, 0)

    def zero_body(i, carry):
        for c in range(_C // 16):
            rw2[i, pl.ds(c * 16, 16)] = jnp.zeros((16,), jnp.float32)
        return carry

    lax.fori_loop(0, _B, zero_body, 0)
    full, rem = _RPS // _B, _RPS % _B
    for k in range(full):
        pltpu.sync_copy(rw2, acc_sh.at[pl.ds(sid * _RPS + k * _B, _B)])
    if rem:
        pltpu.sync_copy(rw2.at[pl.ds(0, rem)],
                        acc_sh.at[pl.ds(sid * _RPS + full * _B, rem)])

    @pl.when(sid == _NS - 1)
    def _zero_tail():
        pltpu.sync_copy(rw2.at[pl.ds(0, _RTAIL)],
                        acc_sh.at[pl.ds(_RPS * _NS, _RTAIL)])

    plsc.subcore_barrier()

    slot(0, 0, 2, scat_wait=False)
    slot(1, 1, 0)
    slot(2, 2, 1)

    def body(k, carry):
        j = 3 * k
        slot(j, 0, 2)
        slot(j + 1, 1, 0)
        slot(j + 2, 2, 1)
        return carry

    lax.fori_loop(1, (_NCHUNK - 6) // 3, body, 0)

    slot(84, 0, 2)
    slot(85, 1, 0)
    slot(86, 2, 1, do_te=False)
    slot(87, 0, 2, do_te=False)
    slot(88, 1, 0, do_gather=False, do_te=False)
    slot(89, 2, 1, do_gather=False, do_te=False)
    wait_scatter(2)

    plsc.subcore_barrier()

    start = sid * _RPS
    pltpu.sync_copy(acc_sh.at[pl.ds(start, _RPS)],
                    out_hbm.at[pl.ds(core * _N + start, _RPS)])

    @pl.when(sid == _NS - 1)
    def _write_tail():
        pltpu.sync_copy(acc_sh.at[pl.ds(_RPS * _NS, _RTAIL)],
                        out_hbm.at[pl.ds(core * _N + _RPS * _NS, _RTAIL)])


_BLK = 2000


def _scale_body(emb_ref, w_ref, o_ref):
    o_ref[...] = emb_ref[...][None, :, :] * w_ref[...][:, None, :]


def _scale(emb, w):
    return pl.pallas_call(
        _scale_body,
        grid=(_N // _BLK,),
        in_specs=[
            pl.BlockSpec((_BLK, _C), lambda i: (i, 0)),
            pl.BlockSpec((_R, _C), lambda i: (0, 0)),
        ],
        out_specs=pl.BlockSpec((_R, _BLK, _C), lambda i: (0, i, 0)),
        out_shape=jax.ShapeDtypeStruct((_R, _N, _C), jnp.float32),
    )(emb, w)


def _combine_body(a_ref, b_ref, o_ref):
    o_ref[...] = a_ref[...] + b_ref[...]


_CBLK = 2000


def _combine(a, b):
    return pl.pallas_call(
        _combine_body,
        grid=(_N // _CBLK,),
        in_specs=[pl.BlockSpec((_CBLK, _C), lambda i: (i, 0))] * 2,
        out_specs=pl.BlockSpec((_CBLK, _C), lambda i: (i, 0)),
        out_shape=jax.ShapeDtypeStruct((_N, _C), jnp.float32),
    )(a, b)


def _pad_per_worker(x, fill):
    return jnp.pad(x.reshape(_NW, _EPW), ((0, 0), (0, _EPAD)),
                   constant_values=fill)


def kernel(all_emb, edge_index, edge_type, weight):
    scaled = _scale(all_emb, weight).reshape(_R * _N, _C)
    tail2 = _pad_per_worker(edge_index[1], 0).reshape(_NW, _NCHUNK, 1, _B)
    et2 = _pad_per_worker(edge_type, 0).reshape(_NW, _NCHUNK, 1, _B)
    head2 = _pad_per_worker(edge_index[0], _N).reshape(_NW, _NCHUNK, 1, _B)
    te = jnp.concatenate([tail2, et2, head2], axis=2)
    partial = _sc_aggregate(scaled, te)
    return _combine(partial[:_N], partial[_N:])


# speedup vs baseline: 1.1884x; 1.0263x over previous
import functools

import jax
import jax.numpy as jnp
from jax import lax
from jax.experimental import pallas as pl
from jax.experimental.pallas import tpu as pltpu
from jax.experimental.pallas import tpu_sc as plsc

_N = 10000
_E = 320000
_C = 128
_R = 10

_NC = 2
_NS = 16
_NW = _NC * _NS
_EPW = _E // _NW
_B = 112
_NCHUNK = -(-_EPW // _B)
_EPAD = _NCHUNK * _B - _EPW
_RPS = 624
_RTAIL = _N - _RPS * _NS

_mesh = plsc.VectorSubcoreMesh(core_axis_name="c", subcore_axis_name="s")


@functools.partial(
    pl.kernel,
    out_type=jax.ShapeDtypeStruct((_NC * _N, _C), jnp.float32),
    mesh=_mesh,
    scratch_types=[
        pltpu.VMEM((3, _B), jnp.int32),
        pltpu.VMEM((3, _B), jnp.int32),
        pltpu.VMEM((3, _B), jnp.int32),
        pltpu.VMEM((_B,), jnp.int32),
        pltpu.VMEM((_B,), jnp.int32),
        pltpu.VMEM((_B,), jnp.int32),
        pltpu.VMEM((_B,), jnp.int32),
        pltpu.VMEM((_B,), jnp.int32),
        pltpu.VMEM((_B,), jnp.int32),
        pltpu.VMEM((_B, _C), jnp.float32),
        pltpu.VMEM((_B, _C), jnp.float32),
        pltpu.VMEM((_B, _C), jnp.float32),
        pltpu.VMEM_SHARED((_N + 8, _C), jnp.float32),
        pltpu.SemaphoreType.DMA,
        pltpu.SemaphoreType.DMA,
        pltpu.SemaphoreType.DMA,
        pltpu.SemaphoreType.DMA,
        pltpu.SemaphoreType.DMA,
        pltpu.SemaphoreType.DMA,
        pltpu.SemaphoreType.DMA,
        pltpu.SemaphoreType.DMA,
        pltpu.SemaphoreType.DMA,
    ],
)
def _sc_aggregate(scaled_hbm, te_hbm, out_hbm,
                  te0, te1, te2, ci0, ci1, ci2, hb0, hb1, hb2,
                  rw0, rw1, rw2, acc_sh,
                  ts0, ts1, ts2, gs0, gs1, gs2, ss0, ss1, ss2):
    core = lax.axis_index("c")
    sid = lax.axis_index("s")
    wid = sid * _NC + core

    te = (te0, te1, te2)
    ci = (ci0, ci1, ci2)
    hb = (hb0, hb1, hb2)
    rw = (rw0, rw1, rw2)
    ts = (ts0, ts1, ts2)
    gs = (gs0, gs1, gs2)
    ss = (ss0, ss1, ss2)

    def issue_te(jx, q):
        pltpu.async_copy(te_hbm.at[wid, jx], te[q], ts[q])

    def wait_te(jx, q):
        pltpu.make_async_copy(te_hbm.at[wid, jx], te[q], ts[q]).wait()

    def prep_idx(q):
        for g in range(_B // 16):
            sl = pl.ds(g * 16, 16)
            ci[q][sl] = te[q][1, sl] * _N + te[q][0, sl]
            hb[q][sl] = te[q][2, sl]

    def issue_gather(q):
        pltpu.async_copy(scaled_hbm.at[ci[q]], rw[q], gs[q])

    def wait_gather(q):
        pltpu.make_async_copy(scaled_hbm.at[ci[q]], rw[q], gs[q]).wait()

    def issue_scatter(q):
        pltpu.async_copy(rw[q], acc_sh.at[hb[q]], ss[q], add=True)

    def wait_scatter(q):
        pltpu.make_async_copy(rw[q], acc_sh.at[hb[q]], ss[q]).wait()

    def slot(j, p, q, scat_wait=True, do_gather=True, do_te=True):
        wait_gather(p)
        issue_scatter(p)
        if do_gather:
            wait_te(j + 2, q)
        if scat_wait:
            wait_scatter(q)
        if do_gather:
            prep_idx(q)
            issue_gather(q)
        if do_te:
            issue_te(j + 4, (p + 1) % 3)

    pltpu.sync_copy(te_hbm.at[wid, 0], te0)
    pltpu.sync_copy(te_hbm.at[wid, 1], te1)
    prep_idx(0)
    prep_idx(1)
    issue_gather(0)
    issue_gather(1)
    issue_te(2, 2)
    issue_te(3, 0)

    def zero_body(i, carry):
        for c in range(_C // 16):
            rw2[i, pl.ds(c * 16, 16)] = jnp.zeros((16,), jnp.float32)
        return carry

    lax.fori_loop(0, _B, zero_body, 0)
    full, rem = _RPS // _B, _RPS % _B
    for k in range(full):
        pltpu.sync_copy(rw2, acc_sh.at[pl.ds(sid * _RPS + k * _B, _B)])
    if rem:
        pltpu.sync_copy(rw2.at[pl.ds(0, rem)],
                        acc_sh.at[pl.ds(sid * _RPS + full * _B, rem)])

    @pl.when(sid == _NS - 1)
    def _zero_tail():
        pltpu.sync_copy(rw2.at[pl.ds(0, _RTAIL)],
                        acc_sh.at[pl.ds(_RPS * _NS, _RTAIL)])

    plsc.subcore_barrier()

    slot(0, 0, 2, scat_wait=False)
    slot(1, 1, 0)
    slot(2, 2, 1)

    def body(k, carry):
        j = 3 * k
        slot(j, 0, 2)
        slot(j + 1, 1, 0)
        slot(j + 2, 2, 1)
        return carry

    lax.fori_loop(1, (_NCHUNK - 6) // 3, body, 0)

    slot(84, 0, 2)
    slot(85, 1, 0)
    slot(86, 2, 1, do_te=False)
    slot(87, 0, 2, do_te=False)
    slot(88, 1, 0, do_gather=False, do_te=False)
    slot(89, 2, 1, do_gather=False, do_te=False)
    wait_scatter(2)

    plsc.subcore_barrier()

    start = sid * _RPS
    pltpu.sync_copy(acc_sh.at[pl.ds(start, _RPS)],
                    out_hbm.at[pl.ds(core * _N + start, _RPS)])

    @pl.when(sid == _NS - 1)
    def _write_tail():
        pltpu.sync_copy(acc_sh.at[pl.ds(_RPS * _NS, _RTAIL)],
                        out_hbm.at[pl.ds(core * _N + _RPS * _NS, _RTAIL)])


_BLK = 2000


def _scale_body(emb_ref, w_ref, o_ref):
    o_ref[...] = emb_ref[...][None, :, :] * w_ref[...][:, None, :]


def _scale(emb, w):
    return pl.pallas_call(
        _scale_body,
        grid=(_N // _BLK,),
        in_specs=[
            pl.BlockSpec((_BLK, _C), lambda i: (i, 0)),
            pl.BlockSpec((_R, _C), lambda i: (0, 0)),
        ],
        out_specs=pl.BlockSpec((_R, _BLK, _C), lambda i: (0, i, 0)),
        out_shape=jax.ShapeDtypeStruct((_R, _N, _C), jnp.float32),
    )(emb, w)


def _combine_body(a_ref, b_ref, o_ref):
    o_ref[...] = a_ref[...] + b_ref[...]


_CBLK = 2000


def _combine(partial):
    return pl.pallas_call(
        _combine_body,
        grid=(_N // _CBLK,),
        in_specs=[
            pl.BlockSpec((_CBLK, _C), lambda i: (i, 0)),
            pl.BlockSpec((_CBLK, _C), lambda i: (i + _N // _CBLK, 0)),
        ],
        out_specs=pl.BlockSpec((_CBLK, _C), lambda i: (i, 0)),
        out_shape=jax.ShapeDtypeStruct((_N, _C), jnp.float32),
    )(partial, partial)


def _pad_per_worker(x, fill):
    return jnp.pad(x.reshape(_NW, _EPW), ((0, 0), (0, _EPAD)),
                   constant_values=fill)


def kernel(all_emb, edge_index, edge_type, weight):
    scaled = _scale(all_emb, weight).reshape(_R * _N, _C)
    tail2 = _pad_per_worker(edge_index[1], 0).reshape(_NW, _NCHUNK, 1, _B)
    et2 = _pad_per_worker(edge_type, 0).reshape(_NW, _NCHUNK, 1, _B)
    head2 = _pad_per_worker(edge_index[0], _N).reshape(_NW, _NCHUNK, 1, _B)
    te = jnp.concatenate([tail2, et2, head2], axis=2)
    partial = _sc_aggregate(scaled, te)
    return _combine(partial)


# speedup vs baseline: 1.1884x; 1.0000x over previous
import functools

import jax
import jax.numpy as jnp
from jax import lax
from jax.experimental import pallas as pl
from jax.experimental.pallas import tpu as pltpu
from jax.experimental.pallas import tpu_sc as plsc

_N = 10000
_E = 320000
_C = 128
_R = 10

_NC = 2
_NS = 16
_NW = _NC * _NS
_EPW = _E // _NW
_B = 112
_NCHUNK = -(-_EPW // _B)
_EPAD = _NCHUNK * _B - _EPW
_RPS = 624
_RTAIL = _N - _RPS * _NS

_mesh = plsc.VectorSubcoreMesh(core_axis_name="c", subcore_axis_name="s")


@functools.partial(
    pl.kernel,
    out_type=jax.ShapeDtypeStruct((_NC * _N, _C), jnp.float32),
    mesh=_mesh,
    scratch_types=[
        pltpu.VMEM((3, _B), jnp.int32),
        pltpu.VMEM((3, _B), jnp.int32),
        pltpu.VMEM((3, _B), jnp.int32),
        pltpu.VMEM((_B,), jnp.int32),
        pltpu.VMEM((_B,), jnp.int32),
        pltpu.VMEM((_B,), jnp.int32),
        pltpu.VMEM((_B,), jnp.int32),
        pltpu.VMEM((_B,), jnp.int32),
        pltpu.VMEM((_B,), jnp.int32),
        pltpu.VMEM((_B, _C), jnp.float32),
        pltpu.VMEM((_B, _C), jnp.float32),
        pltpu.VMEM((_B, _C), jnp.float32),
        pltpu.VMEM_SHARED((_N + 8, _C), jnp.float32),
        pltpu.SemaphoreType.DMA,
        pltpu.SemaphoreType.DMA,
        pltpu.SemaphoreType.DMA,
        pltpu.SemaphoreType.DMA,
        pltpu.SemaphoreType.DMA,
        pltpu.SemaphoreType.DMA,
        pltpu.SemaphoreType.DMA,
        pltpu.SemaphoreType.DMA,
        pltpu.SemaphoreType.DMA,
    ],
)
def _sc_aggregate(scaled_hbm, te_hbm, out_hbm,
                  te0, te1, te2, ci0, ci1, ci2, hb0, hb1, hb2,
                  rw0, rw1, rw2, acc_sh,
                  ts0, ts1, ts2, gs0, gs1, gs2, ss0, ss1, ss2):
    core = lax.axis_index("c")
    sid = lax.axis_index("s")
    wid = sid * _NC + core

    te = (te0, te1, te2)
    ci = (ci0, ci1, ci2)
    hb = (hb0, hb1, hb2)
    rw = (rw0, rw1, rw2)
    ts = (ts0, ts1, ts2)
    gs = (gs0, gs1, gs2)
    ss = (ss0, ss1, ss2)

    def issue_te(jx, q):
        pltpu.async_copy(te_hbm.at[wid, jx], te[q], ts[q])

    def wait_te(jx, q):
        pltpu.make_async_copy(te_hbm.at[wid, jx], te[q], ts[q]).wait()

    def prep_idx(q):
        for g in range(_B // 16):
            sl = pl.ds(g * 16, 16)
            ci[q][sl] = te[q][1, sl] * _N + te[q][0, sl]
            hb[q][sl] = te[q][2, sl]

    def issue_gather(q):
        pltpu.async_copy(scaled_hbm.at[ci[q]], rw[q], gs[q])

    def wait_gather(q):
        pltpu.make_async_copy(scaled_hbm.at[ci[q]], rw[q], gs[q]).wait()

    def issue_scatter(q):
        pltpu.async_copy(rw[q], acc_sh.at[hb[q]], ss[q], add=True)

    def wait_scatter(q):
        pltpu.make_async_copy(rw[q], acc_sh.at[hb[q]], ss[q]).wait()

    def slot(j, p, q, scat_wait=True, do_gather=True, do_te=True):
        wait_gather(p)
        issue_scatter(p)
        if do_gather:
            wait_te(j + 2, q)
        if scat_wait:
            wait_scatter(q)
        if do_gather:
            prep_idx(q)
            issue_gather(q)
        if do_te:
            issue_te(j + 4, (p + 1) % 3)

    pltpu.sync_copy(te_hbm.at[wid, 0], te0)
    pltpu.sync_copy(te_hbm.at[wid, 1], te1)
    prep_idx(0)
    prep_idx(1)
    issue_gather(0)
    issue_gather(1)
    issue_te(2, 2)
    issue_te(3, 0)

    def zero_body(i, carry):
        for c in range(_C // 16):
            rw2[i, pl.ds(c * 16, 16)] = jnp.zeros((16,), jnp.float32)
        return carry

    lax.fori_loop(0, _B, zero_body, 0)
    full, rem = _RPS // _B, _RPS % _B
    for k in range(full):
        pltpu.sync_copy(rw2, acc_sh.at[pl.ds(sid * _RPS + k * _B, _B)])
    if rem:
        pltpu.sync_copy(rw2.at[pl.ds(0, rem)],
                        acc_sh.at[pl.ds(sid * _RPS + full * _B, rem)])

    @pl.when(sid == _NS - 1)
    def _zero_tail():
        pltpu.sync_copy(rw2.at[pl.ds(0, _RTAIL)],
                        acc_sh.at[pl.ds(_RPS * _NS, _RTAIL)])

    plsc.subcore_barrier()

    slot(0, 0, 2, scat_wait=False)
    slot(1, 1, 0)
    slot(2, 2, 1)

    def body(k, carry):
        j = 3 * k
        slot(j, 0, 2)
        slot(j + 1, 1, 0)
        slot(j + 2, 2, 1)
        return carry

    lax.fori_loop(1, (_NCHUNK - 6) // 3, body, 0)

    slot(84, 0, 2)
    slot(85, 1, 0)
    slot(86, 2, 1, do_te=False)
    slot(87, 0, 2, do_te=False)
    slot(88, 1, 0, do_gather=False, do_te=False)
    slot(89, 2, 1, do_gather=False, do_te=False)
    wait_scatter(2)

    plsc.subcore_barrier()

    start = sid * _RPS
    pltpu.sync_copy(acc_sh.at[pl.ds(start, _RPS)],
                    out_hbm.at[pl.ds(core * _N + start, _RPS)])

    @pl.when(sid == _NS - 1)
    def _write_tail():
        pltpu.sync_copy(acc_sh.at[pl.ds(_RPS * _NS, _RTAIL)],
                        out_hbm.at[pl.ds(core * _N + _RPS * _NS, _RTAIL)])


_BLK = 1000


def _scale_body(emb_ref, w_ref, o_ref):
    o_ref[...] = emb_ref[...][None, :, :] * w_ref[...][:, None, :]


def _scale(emb, w):
    return pl.pallas_call(
        _scale_body,
        grid=(_N // _BLK,),
        in_specs=[
            pl.BlockSpec((_BLK, _C), lambda i: (i, 0)),
            pl.BlockSpec((_R, _C), lambda i: (0, 0)),
        ],
        out_specs=pl.BlockSpec((_R, _BLK, _C), lambda i: (0, i, 0)),
        out_shape=jax.ShapeDtypeStruct((_R, _N, _C), jnp.float32),
    )(emb, w)


def _combine_body(a_ref, b_ref, o_ref):
    o_ref[...] = a_ref[...] + b_ref[...]


_CBLK = 2000


def _combine(partial):
    return pl.pallas_call(
        _combine_body,
        grid=(_N // _CBLK,),
        in_specs=[
            pl.BlockSpec((_CBLK, _C), lambda i: (i, 0)),
            pl.BlockSpec((_CBLK, _C), lambda i: (i + _N // _CBLK, 0)),
        ],
        out_specs=pl.BlockSpec((_CBLK, _C), lambda i: (i, 0)),
        out_shape=jax.ShapeDtypeStruct((_N, _C), jnp.float32),
    )(partial, partial)


def _pad_per_worker(x, fill):
    return jnp.pad(x.reshape(_NW, _EPW), ((0, 0), (0, _EPAD)),
                   constant_values=fill)


def kernel(all_emb, edge_index, edge_type, weight):
    scaled = _scale(all_emb, weight).reshape(_R * _N, _C)
    tail2 = _pad_per_worker(edge_index[1], 0).reshape(_NW, _NCHUNK, 1, _B)
    et2 = _pad_per_worker(edge_type, 0).reshape(_NW, _NCHUNK, 1, _B)
    head2 = _pad_per_worker(edge_index[0], _N).reshape(_NW, _NCHUNK, 1, _B)
    te = jnp.concatenate([tail2, et2, head2], axis=2)
    partial = _sc_aggregate(scaled, te)
    return _combine(partial)
